# baseline (speedup 1.0000x reference)
import functools

import jax
import jax.numpy as jnp
from jax import lax
from jax.experimental import pallas as pl
from jax.experimental.pallas import tpu as pltpu
from jax.experimental.pallas import tpu_sc as plsc

B = 128 * 16 * 16 * 16
D = 32
NC = 2
NS = 16
NW = NC * NS
NG = 64
NSTEP = 16

_mesh = plsc.VectorSubcoreMesh(core_axis_name="c", subcore_axis_name="s")


@functools.partial(
    pl.kernel,
    mesh=_mesh,
    compiler_params=pltpu.CompilerParams(use_tc_tiling_on_sc=False),
    out_type=jax.ShapeDtypeStruct((B // 16, 16, D), jnp.float32),
    scratch_types=[
        pltpu.VMEM((4, 16, 16, 16), jnp.int32),
        pltpu.VMEM((2, NG, 16, D), jnp.float32),
        pltpu.SemaphoreType.DMA,
        pltpu.SemaphoreType.DMA,
    ],
)
def _gather(idx_hbm, table_hbm, out_hbm, idx_v, rbuf, gsem, osem):
    wid = lax.axis_index("s") * NC + lax.axis_index("c")
    pltpu.sync_copy(idx_hbm.at[pl.ds(wid * 4, 4)], idx_v)

    def fire_gathers(s, buf):
        bi, x0 = s // 4, (s % 4) * 4
        hs = []
        for g in range(NG):
            xi, yi = x0 + g // 16, g % 16
            hs.append(
                pltpu.async_copy(
                    table_hbm.at[idx_v.at[bi, xi, yi]],
                    buf.at[g],
                    gsem,
                )
            )
        return hs

    gh = fire_gathers(0, rbuf.at[0])
    wh = {}
    for s in range(NSTEP):
        cur = rbuf.at[s % 2]
        if s + 1 < NSTEP:
            if s >= 1:
                wh[s - 1].wait()
            nxt_gh = fire_gathers(s + 1, rbuf.at[(s + 1) % 2])
        for h in gh:
            h.wait()
        wh[s] = pltpu.async_copy(
            cur, out_hbm.at[pl.ds(wid * NSTEP * NG + s * NG, NG)], osem
        )
        if s + 1 < NSTEP:
            gh = nxt_gh
    wh[NSTEP - 2].wait()
    wh[NSTEP - 1].wait()


def kernel(inputs, table):
    z = _gather(inputs, table)
    b, x, y = inputs.shape[0], inputs.shape[1], inputs.shape[2]
    return z.reshape(b, x, y, 16 * D)
